# baseline (speedup 1.0000x reference)
import jax
import jax.numpy as jnp
from jax import lax
from jax.experimental import pallas as pl
from jax.experimental.pallas import tpu as pltpu
from jax.experimental.pallas import tpu_sc as plsc

N = 10000
NP = 10240
E = 160000
EP = 163840
NCH = EP // 128
D = 256
HD = 128
DOUT = 16
RB = 256
GRID = NP // RB

NSUB = 16
ROWS_PER_TILE = NP // NSUB


def _sc_mesh():
    return plsc.VectorSubcoreMesh(
        core_axis_name="c", subcore_axis_name="s", num_cores=2, num_subcores=NSUB
    )


def _deg_body(dst_hbm, ones_hbm, zeros_hbm, out_hbm, didx, ones_v, acc):
    c = lax.axis_index("c")
    s = lax.axis_index("s")
    wid = c * NSUB + s
    r0 = s * ROWS_PER_TILE
    pltpu.sync_copy(zeros_hbm.at[pl.ds(r0, ROWS_PER_TILE)],
                    acc.at[pl.ds(r0, ROWS_PER_TILE)])
    pltpu.sync_copy(ones_hbm, ones_v)
    nch_w = NCH // 32
    pltpu.sync_copy(dst_hbm.at[pl.ds(wid * nch_w, nch_w)], didx)
    plsc.subcore_barrier()

    def body(j, carry):
        pltpu.sync_copy(ones_v, acc.at[didx.at[j]], add=True)
        return carry

    lax.fori_loop(0, nch_w, body, 0)
    plsc.subcore_barrier()
    pltpu.sync_copy(acc.at[pl.ds(r0, ROWS_PER_TILE)],
                    out_hbm.at[pl.ds(c * NP + r0, ROWS_PER_TILE)])


def _deg_call(dstp, ones128, zeros128):
    out = pl.kernel(
        _deg_body,
        out_type=jax.ShapeDtypeStruct((2 * NP, HD), jnp.float32),
        mesh=_sc_mesh(),
        scratch_types=[
            pltpu.VMEM((NCH // 32, 128), jnp.int32),
            pltpu.VMEM((128, HD), jnp.float32),
            pltpu.VMEM_SHARED((NP, HD), jnp.float32),
        ],
    )(dstp, ones128, zeros128)
    return out.reshape(2, NP, HD)


def _seg_body(y_hbm, src2_hbm, dst_hbm, out_hbm, sidx, didx,
              rows0, rows1, sem0, sem1, scsem0, scsem1, acc):
    c = lax.axis_index("c")
    s = lax.axis_index("s")
    r0 = s * ROWS_PER_TILE
    nch_t = NCH // NSUB

    pltpu.sync_copy(y_hbm.at[pl.ds(c * NP + r0, ROWS_PER_TILE)],
                    acc.at[pl.ds(r0, ROWS_PER_TILE)])
    plsc.subcore_barrier()

    PH = 16
    rows = (rows0, rows1)
    sems = (sem0, sem1)
    scsems = (scsem0, scsem1)

    def _drain_scatter(b):
        pltpu.make_async_copy(rows[b], acc.at[pl.ds(0, 128)],
                              scsems[b]).wait()

    def phase(p, carry):
        base = s * nch_t + p * PH
        pltpu.sync_copy(src2_hbm.at[pl.ds(c * NCH + base, PH)], sidx)
        pltpu.sync_copy(dst_hbm.at[pl.ds(base, PH)], didx)
        pltpu.async_copy(y_hbm.at[sidx.at[0]], rows0, sem0)
        for k in range(PH):
            b = k % 2
            if k + 1 < PH:
                if k >= 1:
                    _drain_scatter(1 - b)
                pltpu.async_copy(y_hbm.at[sidx.at[k + 1]], rows[1 - b],
                                 sems[1 - b])
            pltpu.make_async_copy(y_hbm.at[pl.ds(0, 128)], rows[b],
                                  sems[b]).wait()
            pltpu.async_copy(rows[b], acc.at[didx.at[k]], scsems[b], add=True)
        _drain_scatter(PH % 2)
        _drain_scatter(1 - PH % 2)
        return carry

    lax.fori_loop(0, nch_t // PH, phase, 0)
    plsc.subcore_barrier()
    pltpu.sync_copy(acc.at[pl.ds(r0, ROWS_PER_TILE)],
                    out_hbm.at[pl.ds(c * NP + r0, ROWS_PER_TILE)])


def _seg_call(ycat, src2, dstp):
    out = pl.kernel(
        _seg_body,
        out_type=jax.ShapeDtypeStruct((2 * NP, HD), jnp.float32),
        mesh=_sc_mesh(),
        scratch_types=[
            pltpu.VMEM((16, 128), jnp.int32),
            pltpu.VMEM((16, 128), jnp.int32),
            pltpu.VMEM((128, HD), jnp.float32),
            pltpu.VMEM((128, HD), jnp.float32),
            pltpu.SemaphoreType.DMA,
            pltpu.SemaphoreType.DMA,
            pltpu.SemaphoreType.DMA,
            pltpu.SemaphoreType.DMA,
            pltpu.VMEM_SHARED((NP, HD), jnp.float32),
        ],
    )(ycat, src2, dstp)
    return out.reshape(2, NP, HD)


def _dinv_of(d2_ref):
    deg = d2_ref[0, :, 0:1] + d2_ref[1, :, 0:1] + 1.0
    return lax.rsqrt(deg)


def _tc0_body(x_ref, l1w_ref, l1b_ref, l1_ref):
    l1_ref[...] = jnp.maximum(
        jnp.dot(x_ref[...], l1w_ref[...], preferred_element_type=jnp.float32)
        + l1b_ref[...], 0.0)


def _tc0_call(xp, lin1_W, lin1_b2):
    return pl.pallas_call(
        _tc0_body,
        grid=(GRID,),
        in_specs=[
            pl.BlockSpec((RB, D), lambda i: (i, 0)),
            pl.BlockSpec((D, D), lambda i: (0, 0)),
            pl.BlockSpec((1, D), lambda i: (0, 0)),
        ],
        out_specs=pl.BlockSpec((RB, D), lambda i: (i, 0)),
        out_shape=jax.ShapeDtypeStruct((NP, D), jnp.float32),
    )(xp, lin1_W, lin1_b2)


def _tc1_body(x_ref, d2_ref, w1_ref, y1_ref):
    dinv = _dinv_of(d2_ref)
    y = jnp.dot(x_ref[...], w1_ref[...],
                preferred_element_type=jnp.float32) * dinv
    y1_ref[0] = y[:, :HD]
    y1_ref[1] = y[:, HD:]


def _tc1_call(xp, deg2, gcn1_W):
    y1 = pl.pallas_call(
        _tc1_body,
        grid=(GRID,),
        in_specs=[
            pl.BlockSpec((RB, D), lambda i: (i, 0)),
            pl.BlockSpec((2, RB, HD), lambda i: (0, i, 0)),
            pl.BlockSpec((D, D), lambda i: (0, 0)),
        ],
        out_specs=pl.BlockSpec((2, RB, HD), lambda i: (0, i, 0)),
        out_shape=jax.ShapeDtypeStruct((2, NP, HD), jnp.float32),
    )(xp, deg2, gcn1_W)
    return y1.reshape(2 * NP, HD)


def _tc2_body(z_ref, d2_ref, w2_ref, bg1_ref, l1_ref, l2w_ref, l2b_ref,
              wlin_ref, blin_ref, y2_ref, hl_ref):
    dinv = _dinv_of(d2_ref)
    z = jnp.concatenate([z_ref[0], z_ref[1]], axis=1)
    g = jnp.maximum(z * dinv + bg1_ref[...], 0.0)
    y = jnp.dot(g, w2_ref[...], preferred_element_type=jnp.float32) * dinv
    y2_ref[0] = y[:, :HD]
    y2_ref[1] = y[:, HD:]
    l2 = jnp.maximum(
        jnp.dot(l1_ref[...], l2w_ref[...], preferred_element_type=jnp.float32)
        + l2b_ref[...], 0.0)
    hl_ref[...] = (jnp.dot(l2, wlin_ref[...],
                           preferred_element_type=jnp.float32)
                   + blin_ref[...])


def _tc2_call(z1, deg2, gcn2_W, gcn1_b2, l1, lin2_W, lin2_b2, lin_W_l, lin_b2):
    y2, hl = pl.pallas_call(
        _tc2_body,
        grid=(GRID,),
        in_specs=[
            pl.BlockSpec((2, RB, HD), lambda i: (0, i, 0)),
            pl.BlockSpec((2, RB, HD), lambda i: (0, i, 0)),
            pl.BlockSpec((D, D), lambda i: (0, 0)),
            pl.BlockSpec((1, D), lambda i: (0, 0)),
            pl.BlockSpec((RB, D), lambda i: (i, 0)),
            pl.BlockSpec((D, D), lambda i: (0, 0)),
            pl.BlockSpec((1, D), lambda i: (0, 0)),
            pl.BlockSpec((D, DOUT), lambda i: (0, 0)),
            pl.BlockSpec((1, DOUT), lambda i: (0, 0)),
        ],
        out_specs=[
            pl.BlockSpec((2, RB, HD), lambda i: (0, i, 0)),
            pl.BlockSpec((RB, DOUT), lambda i: (i, 0)),
        ],
        out_shape=[
            jax.ShapeDtypeStruct((2, NP, HD), jnp.float32),
            jax.ShapeDtypeStruct((NP, DOUT), jnp.float32),
        ],
    )(z1, deg2, gcn2_W, gcn1_b2, l1, lin2_W, lin2_b2, lin_W_l, lin_b2)
    return y2.reshape(2 * NP, HD), hl


def _tc3_body(z_ref, d2_ref, bg2_ref, hl_ref, wlin_ref, h_ref, p_ref):
    dinv = _dinv_of(d2_ref)
    z = jnp.concatenate([z_ref[0], z_ref[1]], axis=1)
    g2 = jnp.maximum(z * dinv + bg2_ref[...], 0.0)
    h = (jnp.dot(g2, wlin_ref[...], preferred_element_type=jnp.float32)
         + hl_ref[...])
    h_ref[...] = h
    m = jnp.max(h, axis=1, keepdims=True)
    e = jnp.exp(h - m)
    p_ref[...] = e / jnp.sum(e, axis=1, keepdims=True)


def _tc3_call(z2, deg2, gcn2_b2, hl, lin_W_g):
    return pl.pallas_call(
        _tc3_body,
        grid=(GRID,),
        in_specs=[
            pl.BlockSpec((2, RB, HD), lambda i: (0, i, 0)),
            pl.BlockSpec((2, RB, HD), lambda i: (0, i, 0)),
            pl.BlockSpec((1, D), lambda i: (0, 0)),
            pl.BlockSpec((RB, DOUT), lambda i: (i, 0)),
            pl.BlockSpec((D, DOUT), lambda i: (0, 0)),
        ],
        out_specs=[
            pl.BlockSpec((RB, DOUT), lambda i: (i, 0)),
            pl.BlockSpec((RB, DOUT), lambda i: (i, 0)),
        ],
        out_shape=[
            jax.ShapeDtypeStruct((NP, DOUT), jnp.float32),
            jax.ShapeDtypeStruct((NP, DOUT), jnp.float32),
        ],
    )(z2, deg2, gcn2_b2, hl, lin_W_g)


@jax.jit
def kernel(x, edge_index, gcn1_W, gcn1_b, gcn2_W, gcn2_b,
           lin1_W, lin1_b, lin2_W, lin2_b, lin_W, lin_b):
    f32 = jnp.float32
    xp = jnp.zeros((NP, D), f32).at[:N].set(x)
    pad = jnp.full((EP - E,), N, jnp.int32)
    srcp = jnp.concatenate([edge_index[0].astype(jnp.int32), pad]).reshape(NCH, 128)
    dstp = jnp.concatenate([edge_index[1].astype(jnp.int32), pad]).reshape(NCH, 128)
    src2 = jnp.concatenate([srcp, srcp + NP], axis=0)
    ones128 = jnp.ones((128, HD), f32)
    zeros128 = jnp.zeros((NP, HD), f32)

    l1 = _tc0_call(xp, lin1_W, lin1_b.reshape(1, D))
    deg2 = _deg_call(dstp, ones128, zeros128)
    y1 = _tc1_call(xp, deg2, gcn1_W)
    z1 = _seg_call(y1, src2, dstp)
    y2, hl = _tc2_call(z1, deg2, gcn2_W, gcn1_b.reshape(1, D),
                       l1, lin2_W, lin2_b.reshape(1, D),
                       lin_W[:D], lin_b.reshape(1, DOUT))
    z2 = _seg_call(y2, src2, dstp)
    h, p = _tc3_call(z2, deg2, gcn2_b.reshape(1, D), hl, lin_W[D:])
    return (h[:N], p[:N])


# speedup vs baseline: 1.0194x; 1.0194x over previous
import jax
import jax.numpy as jnp
from jax import lax
from jax.experimental import pallas as pl
from jax.experimental.pallas import tpu as pltpu
from jax.experimental.pallas import tpu_sc as plsc

N = 10000
NP = 10240
E = 160000
EP = 163840
NCH = EP // 128
D = 256
HD = 128
DOUT = 16
RB = 256
GRID = NP // RB

NSUB = 16
ROWS_PER_TILE = NP // NSUB


def _sc_mesh():
    return plsc.VectorSubcoreMesh(
        core_axis_name="c", subcore_axis_name="s", num_cores=2, num_subcores=NSUB
    )


def _deg_body(dst_hbm, ones_hbm, zeros_hbm, out_hbm, didx, ones_v, acc):
    c = lax.axis_index("c")
    s = lax.axis_index("s")
    wid = c * NSUB + s
    r0 = s * ROWS_PER_TILE
    pltpu.sync_copy(zeros_hbm.at[pl.ds(r0, ROWS_PER_TILE)],
                    acc.at[pl.ds(r0, ROWS_PER_TILE)])
    pltpu.sync_copy(ones_hbm, ones_v)
    nch_w = NCH // 32
    pltpu.sync_copy(dst_hbm.at[pl.ds(wid * nch_w, nch_w)], didx)
    plsc.subcore_barrier()

    def body(j, carry):
        pltpu.sync_copy(ones_v, acc.at[didx.at[j]], add=True)
        return carry

    lax.fori_loop(0, nch_w, body, 0)
    plsc.subcore_barrier()
    pltpu.sync_copy(acc.at[pl.ds(r0, ROWS_PER_TILE)],
                    out_hbm.at[pl.ds(c * NP + r0, ROWS_PER_TILE)])


def _deg_call(dstp, ones128, zeros128):
    out = pl.kernel(
        _deg_body,
        out_type=jax.ShapeDtypeStruct((2 * NP, HD), jnp.float32),
        mesh=_sc_mesh(),
        scratch_types=[
            pltpu.VMEM((NCH // 32, 128), jnp.int32),
            pltpu.VMEM((128, HD), jnp.float32),
            pltpu.VMEM_SHARED((NP, HD), jnp.float32),
        ],
    )(dstp, ones128, zeros128)
    return out.reshape(2, NP, HD)


def _seg_body(y_hbm, src2_hbm, dst_hbm, out_hbm, sidx, didx,
              rows0, rows1, sem0, sem1, scsem0, scsem1, acc):
    c = lax.axis_index("c")
    s = lax.axis_index("s")
    r0 = s * ROWS_PER_TILE
    nch_t = NCH // NSUB

    pltpu.sync_copy(y_hbm.at[pl.ds(c * NP + r0, ROWS_PER_TILE)],
                    acc.at[pl.ds(r0, ROWS_PER_TILE)])
    plsc.subcore_barrier()

    PH = 16
    rows = (rows0, rows1)
    sems = (sem0, sem1)
    scsems = (scsem0, scsem1)

    def _drain_scatter(b):
        pltpu.make_async_copy(rows[b], acc.at[pl.ds(0, 128)],
                              scsems[b]).wait()

    def phase(p, carry):
        base = s * nch_t + p * PH
        pltpu.sync_copy(src2_hbm.at[pl.ds(c * NCH + base, PH)], sidx)
        pltpu.sync_copy(dst_hbm.at[pl.ds(base, PH)], didx)
        pltpu.async_copy(y_hbm.at[sidx.at[0]], rows0, sem0)
        for k in range(PH):
            b = k % 2
            if k + 1 < PH:
                if k >= 1:
                    _drain_scatter(1 - b)
                pltpu.async_copy(y_hbm.at[sidx.at[k + 1]], rows[1 - b],
                                 sems[1 - b])
            pltpu.make_async_copy(y_hbm.at[pl.ds(0, 128)], rows[b],
                                  sems[b]).wait()
            pltpu.async_copy(rows[b], acc.at[didx.at[k]], scsems[b], add=True)
        _drain_scatter(PH % 2)
        _drain_scatter(1 - PH % 2)
        return carry

    lax.fori_loop(0, nch_t // PH, phase, 0)
    plsc.subcore_barrier()
    pltpu.sync_copy(acc.at[pl.ds(r0, ROWS_PER_TILE)],
                    out_hbm.at[pl.ds(c * NP + r0, ROWS_PER_TILE)])


def _seg_call(ycat, src2, dstp):
    out = pl.kernel(
        _seg_body,
        out_type=jax.ShapeDtypeStruct((2 * NP, HD), jnp.float32),
        mesh=_sc_mesh(),
        scratch_types=[
            pltpu.VMEM((16, 128), jnp.int32),
            pltpu.VMEM((16, 128), jnp.int32),
            pltpu.VMEM((128, HD), jnp.float32),
            pltpu.VMEM((128, HD), jnp.float32),
            pltpu.SemaphoreType.DMA,
            pltpu.SemaphoreType.DMA,
            pltpu.SemaphoreType.DMA,
            pltpu.SemaphoreType.DMA,
            pltpu.VMEM_SHARED((NP, HD), jnp.float32),
        ],
    )(ycat, src2, dstp)
    return out.reshape(2, NP, HD)


def _dinv_of(d2_ref):
    deg = d2_ref[0, :, 0:1] + d2_ref[1, :, 0:1] + 1.0
    return lax.rsqrt(deg)


def _tc0_body(x_ref, l1w_ref, l1b_ref, l1_ref):
    l1_ref[...] = jnp.maximum(
        jnp.dot(x_ref[...], l1w_ref[...], preferred_element_type=jnp.float32)
        + l1b_ref[...], 0.0)


def _tc0_call(xp, lin1_W, lin1_b2):
    return pl.pallas_call(
        _tc0_body,
        grid=(GRID,),
        in_specs=[
            pl.BlockSpec((RB, D), lambda i: (i, 0)),
            pl.BlockSpec((D, D), lambda i: (0, 0)),
            pl.BlockSpec((1, D), lambda i: (0, 0)),
        ],
        out_specs=pl.BlockSpec((RB, D), lambda i: (i, 0)),
        out_shape=jax.ShapeDtypeStruct((NP, D), jnp.float32),
    )(xp, lin1_W, lin1_b2)


def _tc1_body(x_ref, d2_ref, w1_ref, y1_ref):
    dinv = _dinv_of(d2_ref)
    y = jnp.dot(x_ref[...], w1_ref[...],
                preferred_element_type=jnp.float32) * dinv
    y1_ref[0] = y[:, :HD]
    y1_ref[1] = y[:, HD:]


def _tc1_call(xp, deg2, gcn1_W):
    y1 = pl.pallas_call(
        _tc1_body,
        grid=(GRID,),
        in_specs=[
            pl.BlockSpec((RB, D), lambda i: (i, 0)),
            pl.BlockSpec((2, RB, HD), lambda i: (0, i, 0)),
            pl.BlockSpec((D, D), lambda i: (0, 0)),
        ],
        out_specs=pl.BlockSpec((2, RB, HD), lambda i: (0, i, 0)),
        out_shape=jax.ShapeDtypeStruct((2, NP, HD), jnp.float32),
    )(xp, deg2, gcn1_W)
    return y1.reshape(2 * NP, HD)


def _tc2_body(z_ref, d2_ref, w2_ref, bg1_ref, y2_ref):
    dinv = _dinv_of(d2_ref)
    z = jnp.concatenate([z_ref[0], z_ref[1]], axis=1)
    g = jnp.maximum(z * dinv + bg1_ref[...], 0.0)
    y = jnp.dot(g, w2_ref[...], preferred_element_type=jnp.float32) * dinv
    y2_ref[0] = y[:, :HD]
    y2_ref[1] = y[:, HD:]


def _tc2_call(z1, deg2, gcn2_W, gcn1_b2):
    y2 = pl.pallas_call(
        _tc2_body,
        grid=(GRID,),
        in_specs=[
            pl.BlockSpec((2, RB, HD), lambda i: (0, i, 0)),
            pl.BlockSpec((2, RB, HD), lambda i: (0, i, 0)),
            pl.BlockSpec((D, D), lambda i: (0, 0)),
            pl.BlockSpec((1, D), lambda i: (0, 0)),
        ],
        out_specs=pl.BlockSpec((2, RB, HD), lambda i: (0, i, 0)),
        out_shape=jax.ShapeDtypeStruct((2, NP, HD), jnp.float32),
    )(z1, deg2, gcn2_W, gcn1_b2)
    return y2.reshape(2 * NP, HD)


def _tcl2_body(l1_ref, l2w_ref, l2b_ref, wlin_ref, blin_ref, hl_ref):
    l2 = jnp.maximum(
        jnp.dot(l1_ref[...], l2w_ref[...], preferred_element_type=jnp.float32)
        + l2b_ref[...], 0.0)
    hl_ref[...] = (jnp.dot(l2, wlin_ref[...],
                           preferred_element_type=jnp.float32)
                   + blin_ref[...])


def _tcl2_call(l1, lin2_W, lin2_b2, lin_W_l, lin_b2):
    return pl.pallas_call(
        _tcl2_body,
        grid=(GRID,),
        in_specs=[
            pl.BlockSpec((RB, D), lambda i: (i, 0)),
            pl.BlockSpec((D, D), lambda i: (0, 0)),
            pl.BlockSpec((1, D), lambda i: (0, 0)),
            pl.BlockSpec((D, DOUT), lambda i: (0, 0)),
            pl.BlockSpec((1, DOUT), lambda i: (0, 0)),
        ],
        out_specs=pl.BlockSpec((RB, DOUT), lambda i: (i, 0)),
        out_shape=jax.ShapeDtypeStruct((NP, DOUT), jnp.float32),
    )(l1, lin2_W, lin2_b2, lin_W_l, lin_b2)


def _tc3_body(z_ref, d2_ref, bg2_ref, hl_ref, wlin_ref, h_ref, p_ref):
    dinv = _dinv_of(d2_ref)
    z = jnp.concatenate([z_ref[0], z_ref[1]], axis=1)
    g2 = jnp.maximum(z * dinv + bg2_ref[...], 0.0)
    h = (jnp.dot(g2, wlin_ref[...], preferred_element_type=jnp.float32)
         + hl_ref[...])
    h_ref[...] = h
    m = jnp.max(h, axis=1, keepdims=True)
    e = jnp.exp(h - m)
    p_ref[...] = e / jnp.sum(e, axis=1, keepdims=True)


def _tc3_call(z2, deg2, gcn2_b2, hl, lin_W_g):
    return pl.pallas_call(
        _tc3_body,
        grid=(GRID,),
        in_specs=[
            pl.BlockSpec((2, RB, HD), lambda i: (0, i, 0)),
            pl.BlockSpec((2, RB, HD), lambda i: (0, i, 0)),
            pl.BlockSpec((1, D), lambda i: (0, 0)),
            pl.BlockSpec((RB, DOUT), lambda i: (i, 0)),
            pl.BlockSpec((D, DOUT), lambda i: (0, 0)),
        ],
        out_specs=[
            pl.BlockSpec((RB, DOUT), lambda i: (i, 0)),
            pl.BlockSpec((RB, DOUT), lambda i: (i, 0)),
        ],
        out_shape=[
            jax.ShapeDtypeStruct((NP, DOUT), jnp.float32),
            jax.ShapeDtypeStruct((NP, DOUT), jnp.float32),
        ],
    )(z2, deg2, gcn2_b2, hl, lin_W_g)


@jax.jit
def kernel(x, edge_index, gcn1_W, gcn1_b, gcn2_W, gcn2_b,
           lin1_W, lin1_b, lin2_W, lin2_b, lin_W, lin_b):
    f32 = jnp.float32
    xp = jnp.zeros((NP, D), f32).at[:N].set(x)
    pad = jnp.full((EP - E,), N, jnp.int32)
    srcp = jnp.concatenate([edge_index[0].astype(jnp.int32), pad]).reshape(NCH, 128)
    dstp = jnp.concatenate([edge_index[1].astype(jnp.int32), pad]).reshape(NCH, 128)
    src2 = jnp.concatenate([srcp, srcp + NP], axis=0)
    ones128 = jnp.ones((128, HD), f32)
    zeros128 = jnp.zeros((NP, HD), f32)

    l1 = _tc0_call(xp, lin1_W, lin1_b.reshape(1, D))
    deg2 = _deg_call(dstp, ones128, zeros128)
    y1 = _tc1_call(xp, deg2, gcn1_W)
    z1 = _seg_call(y1, src2, dstp)
    y2 = _tc2_call(z1, deg2, gcn2_W, gcn1_b.reshape(1, D))
    hl = _tcl2_call(l1, lin2_W, lin2_b.reshape(1, D),
                    lin_W[:D], lin_b.reshape(1, DOUT))
    z2 = _seg_call(y2, src2, dstp)
    h, p = _tc3_call(z2, deg2, gcn2_b.reshape(1, D), hl, lin_W[D:])
    return (h[:N], p[:N])
